# baseline (speedup 1.0000x reference)
import functools

import jax
import jax.numpy as jnp
from jax import lax
from jax.experimental import pallas as pl
from jax.experimental.pallas import tpu as pltpu
from jax.experimental.pallas import tpu_sc as plsc

D = 128
PATH_LEN = 4096
NUM_CAND = 64

NC = 2
NS = 16
NW = NC * NS
ROWS_PER_TILE = PATH_LEN // NW
NCHUNK = 1
CHUNK = ROWS_PER_TILE // NCHUNK
UNROLL = 2
L = 16
DV = D // L

CAND_TILE = 1
TARG_TILE = 3


def _sc_gather(hex_embed, path_ids, candidate_ids, target_id):
    mesh = plsc.VectorSubcoreMesh(core_axis_name="c", subcore_axis_name="s")

    @functools.partial(
        pl.kernel,
        out_type=(
            jax.ShapeDtypeStruct((NW, D), jnp.float32),
            jax.ShapeDtypeStruct((NUM_CAND + 1, D), jnp.float32),
        ),
        mesh=mesh,
        scratch_types=[
            pltpu.VMEM((ROWS_PER_TILE,), jnp.int32),
            [pltpu.VMEM((CHUNK, D), jnp.float32) for _ in range(NCHUNK)],
            pltpu.VMEM((1, D), jnp.float32),
            pltpu.VMEM((NUM_CAND,), jnp.int32),
            pltpu.VMEM((NUM_CAND, D), jnp.float32),
            pltpu.VMEM((1,), jnp.int32),
            pltpu.VMEM((1, D), jnp.float32),
            [pltpu.SemaphoreType.DMA for _ in range(NCHUNK)],
            pltpu.SemaphoreType.DMA,
        ],
    )
    def k(table_hbm, pids_hbm, cids_hbm, tid_hbm,
          partial_out, extra_out,
          idx_v, rows, acc_v, eidx_v, erows_v,
          tidx_v, trow_v, sems, sem_e):
        wid = lax.axis_index("s") * NC + lax.axis_index("c")
        base = wid * ROWS_PER_TILE
        pltpu.sync_copy(pids_hbm.at[pl.ds(base, ROWS_PER_TILE)], idx_v)
        for c in range(NCHUNK):
            pltpu.make_async_copy(
                table_hbm.at[idx_v.at[pl.ds(c * CHUNK, CHUNK)]],
                rows[c], sems[c]).start()

        @pl.when(wid == CAND_TILE)
        def _():
            pltpu.sync_copy(cids_hbm, eidx_v)
            pltpu.make_async_copy(table_hbm.at[eidx_v], erows_v, sem_e).start()

        @pl.when(wid == TARG_TILE)
        def _():
            pltpu.sync_copy(tid_hbm, tidx_v)
            pltpu.make_async_copy(table_hbm.at[tidx_v], trow_v, sem_e).start()

        def accum_chunk(rows_v, acc):
            def body(it, a):
                r0 = it * UNROLL
                for dr in range(UNROLL):
                    a = tuple(a[j] + rows_v[r0 + dr, pl.ds(j * L, L)]
                              for j in range(DV))
                return a
            return lax.fori_loop(0, CHUNK // UNROLL, body, acc)

        acc = tuple(jnp.zeros((L,), jnp.float32) for _ in range(DV))
        for c in range(NCHUNK):
            pltpu.make_async_copy(
                table_hbm.at[idx_v.at[pl.ds(c * CHUNK, CHUNK)]],
                rows[c], sems[c]).wait()
            acc = accum_chunk(rows[c], acc)
        for j in range(DV):
            acc_v[0, pl.ds(j * L, L)] = acc[j]
        pltpu.sync_copy(acc_v, partial_out.at[pl.ds(wid, 1)])

        @pl.when(wid == CAND_TILE)
        def _():
            pltpu.make_async_copy(table_hbm.at[eidx_v], erows_v, sem_e).wait()
            pltpu.sync_copy(erows_v, extra_out.at[pl.ds(0, NUM_CAND)])

        @pl.when(wid == TARG_TILE)
        def _():
            pltpu.make_async_copy(table_hbm.at[tidx_v], trow_v, sem_e).wait()
            pltpu.sync_copy(trow_v, extra_out.at[pl.ds(NUM_CAND, 1)])

    return k(hex_embed, path_ids, candidate_ids, target_id)


def _dense_body(part_ref, ext_ref, wp_ref, bp_ref, wt_ref, bt_ref,
                wo_ref, bo_ref, out_ref):
    path_mean = jnp.sum(part_ref[...], axis=0, keepdims=True) * (1.0 / PATH_LEN)
    dn = (((1,), (1,)), ((), ()))
    combined = (
        lax.dot_general(path_mean, wp_ref[...], dn)
        + lax.dot_general(ext_ref[NUM_CAND:NUM_CAND + 1, :], wt_ref[...], dn)
        + bp_ref[...][None, :] + bt_ref[...][None, :])
    h = jnp.maximum(ext_ref[0:NUM_CAND, :] + combined, 0.0)
    scores = jnp.sum(h * wo_ref[...], axis=1) + bo_ref[...]
    m = jnp.max(scores)
    e = jnp.exp(scores - m)
    out_ref[...] = e * (1.0 / jnp.sum(e))


def _tc_dense(partials, extras, Wp, bp, Wt, bt, Wo, bo):
    return pl.pallas_call(
        _dense_body,
        out_shape=jax.ShapeDtypeStruct((NUM_CAND,), jnp.float32),
    )(partials, extras, Wp, bp, Wt, bt, Wo, bo)


def kernel(path_ids, target_id, candidate_ids, hex_embed, Wp, bp, Wt, bt, Wo, bo):
    partials, extras = _sc_gather(hex_embed, path_ids, candidate_ids, target_id)
    return _tc_dense(partials, extras, Wp, bp, Wt, bt, Wo, bo)


# speedup vs baseline: 1.0090x; 1.0090x over previous
import functools

import jax
import jax.numpy as jnp
from jax import lax
from jax.experimental import pallas as pl
from jax.experimental.pallas import tpu as pltpu
from jax.experimental.pallas import tpu_sc as plsc

D = 128
PATH_LEN = 4096
NUM_CAND = 64

NC = 2
NS = 16
NW = NC * NS
ROWS_PER_TILE = PATH_LEN // NW
NCHUNK = 2
CHUNK = ROWS_PER_TILE // NCHUNK
UNROLL = 2
L = 16
DV = D // L

CAND_TILE = 1
TARG_TILE = 3


def _sc_gather(hex_embed, path_ids, candidate_ids, target_id):
    mesh = plsc.VectorSubcoreMesh(core_axis_name="c", subcore_axis_name="s")

    @functools.partial(
        pl.kernel,
        out_type=(
            jax.ShapeDtypeStruct((NW, D), jnp.float32),
            jax.ShapeDtypeStruct((NUM_CAND + 1, D), jnp.float32),
        ),
        mesh=mesh,
        scratch_types=[
            pltpu.VMEM((ROWS_PER_TILE,), jnp.int32),
            [pltpu.VMEM((CHUNK, D), jnp.float32) for _ in range(NCHUNK)],
            pltpu.VMEM((1, D), jnp.float32),
            pltpu.VMEM((NUM_CAND,), jnp.int32),
            pltpu.VMEM((NUM_CAND, D), jnp.float32),
            pltpu.VMEM((1,), jnp.int32),
            pltpu.VMEM((1, D), jnp.float32),
            [pltpu.SemaphoreType.DMA for _ in range(NCHUNK)],
            pltpu.SemaphoreType.DMA,
        ],
    )
    def k(table_hbm, pids_hbm, cids_hbm, tid_hbm,
          partial_out, extra_out,
          idx_v, rows, acc_v, eidx_v, erows_v,
          tidx_v, trow_v, sems, sem_e):
        wid = lax.axis_index("s") * NC + lax.axis_index("c")
        base = wid * ROWS_PER_TILE
        pltpu.sync_copy(pids_hbm.at[pl.ds(base, ROWS_PER_TILE)], idx_v)
        for c in range(NCHUNK):
            pltpu.make_async_copy(
                table_hbm.at[idx_v.at[pl.ds(c * CHUNK, CHUNK)]],
                rows[c], sems[c]).start()

        @pl.when(wid == CAND_TILE)
        def _():
            pltpu.sync_copy(cids_hbm, eidx_v)
            pltpu.make_async_copy(table_hbm.at[eidx_v], erows_v, sem_e).start()

        @pl.when(wid == TARG_TILE)
        def _():
            pltpu.sync_copy(tid_hbm, tidx_v)
            pltpu.make_async_copy(table_hbm.at[tidx_v], trow_v, sem_e).start()

        def accum_chunk(rows_v, acc):
            def body(it, a):
                r0 = it * UNROLL
                for dr in range(UNROLL):
                    a = tuple(a[j] + rows_v[r0 + dr, pl.ds(j * L, L)]
                              for j in range(DV))
                return a
            return lax.fori_loop(0, CHUNK // UNROLL, body, acc)

        acc = tuple(jnp.zeros((L,), jnp.float32) for _ in range(DV))
        for c in range(NCHUNK):
            pltpu.make_async_copy(
                table_hbm.at[idx_v.at[pl.ds(c * CHUNK, CHUNK)]],
                rows[c], sems[c]).wait()
            acc = accum_chunk(rows[c], acc)
        for j in range(DV):
            acc_v[0, pl.ds(j * L, L)] = acc[j]
        pltpu.sync_copy(acc_v, partial_out.at[pl.ds(wid, 1)])

        @pl.when(wid == CAND_TILE)
        def _():
            pltpu.make_async_copy(table_hbm.at[eidx_v], erows_v, sem_e).wait()
            pltpu.sync_copy(erows_v, extra_out.at[pl.ds(0, NUM_CAND)])

        @pl.when(wid == TARG_TILE)
        def _():
            pltpu.make_async_copy(table_hbm.at[tidx_v], trow_v, sem_e).wait()
            pltpu.sync_copy(trow_v, extra_out.at[pl.ds(NUM_CAND, 1)])

    return k(hex_embed, path_ids, candidate_ids, target_id)


def _dense_body(part_ref, ext_ref, wp_ref, bp_ref, wt_ref, bt_ref,
                wo_ref, bo_ref, out_ref):
    path_mean = jnp.sum(part_ref[...], axis=0, keepdims=True) * (1.0 / PATH_LEN)
    dn = (((1,), (1,)), ((), ()))
    combined = (
        lax.dot_general(path_mean, wp_ref[...], dn)
        + lax.dot_general(ext_ref[NUM_CAND:NUM_CAND + 1, :], wt_ref[...], dn)
        + bp_ref[...][None, :] + bt_ref[...][None, :])
    h = jnp.maximum(ext_ref[0:NUM_CAND, :] + combined, 0.0)
    scores = jnp.sum(h * wo_ref[...], axis=1) + bo_ref[...]
    m = jnp.max(scores)
    e = jnp.exp(scores - m)
    out_ref[...] = e * (1.0 / jnp.sum(e))


def _tc_dense(partials, extras, Wp, bp, Wt, bt, Wo, bo):
    return pl.pallas_call(
        _dense_body,
        out_shape=jax.ShapeDtypeStruct((NUM_CAND,), jnp.float32),
    )(partials, extras, Wp, bp, Wt, bt, Wo, bo)


def kernel(path_ids, target_id, candidate_ids, hex_embed, Wp, bp, Wt, bt, Wo, bo):
    partials, extras = _sc_gather(hex_embed, path_ids, candidate_ids, target_id)
    return _tc_dense(partials, extras, Wp, bp, Wt, bt, Wo, bo)
